# baseline (speedup 1.0000x reference)
import functools

import jax
import jax.numpy as jnp
from jax import lax
from jax.experimental import pallas as pl
from jax.experimental.pallas import tpu as pltpu
from jax.experimental.pallas import tpu_sc as plsc

BATCH = 16384
DIM = 32
LANES = 16
NUM_WORKERS = 32
ROWS_PER_W = BATCH // NUM_WORKERS
CHUNK = 128
NCHUNK = ROWS_PER_W // CHUNK
BLOCKS_PER_CHUNK = CHUNK // LANES


def _sc_body(uidx_hbm, iidx_hbm, utab_hbm, itab_hbm, wb_hbm, b_hbm,
             out_hbm,
             uidx_v, iidx_v, urows_v, irows_v, wb_v, bv_v, outv,
             *sems):
    nc = 2
    wid = lax.axis_index("s") * nc + lax.axis_index("c")
    base = wid * ROWS_PER_W

    pltpu.sync_copy(uidx_hbm.at[pl.ds(wid * NCHUNK, NCHUNK)], uidx_v)
    pltpu.sync_copy(iidx_hbm.at[pl.ds(wid * NCHUNK, NCHUNK)], iidx_v)
    pltpu.sync_copy(wb_hbm, wb_v)
    pltpu.sync_copy(b_hbm, bv_v)

    ucopies = []
    icopies = []
    for j in range(NCHUNK):
        ucopies.append(pltpu.async_copy(
            utab_hbm.at[uidx_v.at[j]],
            urows_v.at[pl.ds(j * CHUNK, CHUNK)], sems[j]))
        icopies.append(pltpu.async_copy(
            itab_hbm.at[iidx_v.at[j]],
            irows_v.at[pl.ds(j * CHUNK, CHUNK)], sems[NCHUNK + j]))

    lanes = lax.iota(jnp.int32, LANES)

    for j in range(NCHUNK):
        ucopies[j].wait()
        icopies[j].wait()

        def block_body(bb, _, j=j):
            rows = (j * BLOCKS_PER_CHUNK + bb) * LANES + lanes
            acc = bv_v[:]
            for d in range(DIM):
                col = jnp.full((LANES,), d, jnp.int32)
                gu = plsc.load_gather(urows_v, [rows, col])
                gi = plsc.load_gather(irows_v, [rows, col])
                acc = acc + gu * gi * wb_v[d, :]
            sig = 1.0 / (1.0 + jnp.exp(-acc))
            outv[pl.ds((j * BLOCKS_PER_CHUNK + bb) * LANES, LANES)] = sig
            return _

        lax.fori_loop(0, BLOCKS_PER_CHUNK, block_body, 0, unroll=False)

    pltpu.sync_copy(outv, out_hbm.at[pl.ds(base, ROWS_PER_W)])


@jax.jit
def _run(uidx2, iidx2, user_table, item_table, wb, b16):
    mesh = plsc.VectorSubcoreMesh(core_axis_name="c", subcore_axis_name="s",
                                  num_cores=2, num_subcores=16)
    scratch = [
        pltpu.VMEM((NCHUNK, CHUNK), jnp.int32),
        pltpu.VMEM((NCHUNK, CHUNK), jnp.int32),
        pltpu.VMEM((ROWS_PER_W, DIM), jnp.float32),
        pltpu.VMEM((ROWS_PER_W, DIM), jnp.float32),
        pltpu.VMEM((DIM, LANES), jnp.float32),
        pltpu.VMEM((LANES,), jnp.float32),
        pltpu.VMEM((ROWS_PER_W,), jnp.float32),
    ] + [pltpu.SemaphoreType.DMA] * (2 * NCHUNK)
    fn = pl.kernel(
        _sc_body,
        out_type=jax.ShapeDtypeStruct((BATCH,), jnp.float32),
        mesh=mesh,
        scratch_types=scratch,
        compiler_params=pltpu.CompilerParams(needs_layout_passes=False,
                                             use_tc_tiling_on_sc=False),
    )
    return fn(uidx2, iidx2, user_table, item_table, wb, b16)


def kernel(user_indices, item_indices, user_table, item_table, W, b):
    uidx2 = user_indices.astype(jnp.int32).reshape(CHUNK, CHUNK)
    iidx2 = item_indices.astype(jnp.int32).reshape(CHUNK, CHUNK)
    wb = jnp.broadcast_to(W.reshape(DIM, 1), (DIM, LANES))
    b16 = jnp.broadcast_to(b.reshape(1), (LANES,)).astype(jnp.float32)
    out = _run(uidx2, iidx2, user_table, item_table, wb, b16)
    return out.reshape(BATCH, 1)


# speedup vs baseline: 3.6761x; 3.6761x over previous
import jax
import jax.numpy as jnp
from jax import lax
from jax.experimental import pallas as pl
from jax.experimental.pallas import tpu as pltpu
from jax.experimental.pallas import tpu_sc as plsc

BATCH = 16384
DIM = 32
LANES = 16
NUM_WORKERS = 32
ROWS_PER_W = BATCH // NUM_WORKERS
NGROUP = ROWS_PER_W // LANES
RING = 4


def _sc_body(uidx_hbm, iidx_hbm, ttu_hbm, tti_hbm, w_hbm, b_hbm, dummy_hbm,
             out_hbm,
             uidx_v, iidx_v, ubufw, ibufw, wv_v, bv_v, tbuf, outv,
             sem0, sem1, sem2, sem3):
    nc = 2
    wid = lax.axis_index("s") * nc + lax.axis_index("c")
    sems = (sem0, sem1, sem2, sem3)

    pltpu.sync_copy(uidx_hbm.at[pl.ds(wid * 4, 4)], uidx_v)
    pltpu.sync_copy(iidx_hbm.at[pl.ds(wid * 4, 4)], iidx_v)
    pltpu.sync_copy(w_hbm, wv_v)
    pltpu.sync_copy(b_hbm, bv_v)

    lanes = lax.iota(jnp.int32, LANES)
    wv0 = wv_v[pl.ds(0, LANES)]
    wv1 = wv_v[pl.ds(LANES, LANES)]

    def issue(slot, ur, ir):
        uc0 = pl.multiple_of((ur // 128) * 128, 128)
        ic0 = pl.multiple_of((ir // 128) * 128, 128)
        pltpu.async_copy(ttu_hbm.at[:, pl.ds(uc0, 128)],
                         ubufw.at[slot], sems[slot])
        pltpu.async_copy(tti_hbm.at[:, pl.ds(ic0, 128)],
                         ibufw.at[slot], sems[slot])

    def finish(slot, f, ur, ir):
        pltpu.make_async_copy(dummy_hbm, ubufw.at[slot], sems[slot]).wait()
        pltpu.make_async_copy(dummy_hbm, ibufw.at[slot], sems[slot]).wait()
        sv = jnp.full((LANES,), slot, jnp.int32)
        cu = jnp.full((LANES,), lax.rem(ur, 128), jnp.int32)
        ci = jnp.full((LANES,), lax.rem(ir, 128), jnp.int32)
        gu0 = plsc.load_gather(ubufw, [sv, lanes, cu])
        gu1 = plsc.load_gather(ubufw, [sv, lanes + LANES, cu])
        gi0 = plsc.load_gather(ibufw, [sv, lanes, ci])
        gi1 = plsc.load_gather(ibufw, [sv, lanes + LANES, ci])
        t = gu0 * gi0 * wv0 + gu1 * gi1 * wv1
        tbuf[f // 8, pl.ds(lax.rem(f, 8) * LANES, LANES)] = t

    def group_body(g, carry):
        puvec, pivec = carry
        f0 = g * LANES
        row = f0 // 128
        col = lax.rem(f0, 128)
        uvec = uidx_v[row, pl.ds(col, LANES)]
        ivec = iidx_v[row, pl.ds(col, LANES)]
        for k in range(LANES):
            slot = k % RING
            if k >= RING:
                finish(slot, f0 + k - RING, uvec[k - RING], ivec[k - RING])
            else:
                @pl.when(g > 0)
                def _fin(slot=slot, k=k, puvec=puvec, pivec=pivec):
                    finish(slot, f0 + k - RING,
                           puvec[12 + k], pivec[12 + k])
            issue(slot, uvec[k], ivec[k])
        return uvec, ivec

    zero16 = jnp.zeros((LANES,), jnp.int32)
    luvec, livec = lax.fori_loop(0, NGROUP, group_body, (zero16, zero16))

    for k in range(RING):
        finish((LANES - RING + k) % RING, ROWS_PER_W - RING + k,
               luvec[LANES - RING + k], livec[LANES - RING + k])

    def sig_body(j, carry):
        rows = j * LANES + lanes
        rr = rows // 8
        rc = lax.rem(rows, 8) * LANES
        x = bv_v[:]
        for l in range(LANES):
            x = x + plsc.load_gather(tbuf, [rr, rc + l])
        outv[pl.ds(j * LANES, LANES)] = 1.0 / (1.0 + jnp.exp(-x))
        return carry

    lax.fori_loop(0, ROWS_PER_W // LANES, sig_body, 0)

    pltpu.sync_copy(outv, out_hbm.at[pl.ds(wid * ROWS_PER_W, ROWS_PER_W)])


@jax.jit
def _run(uidx2, iidx2, ttu, tti, w32, b16, dummy):
    mesh = plsc.VectorSubcoreMesh(core_axis_name="c", subcore_axis_name="s",
                                  num_cores=2, num_subcores=16)
    scratch = [
        pltpu.VMEM((4, 128), jnp.int32),
        pltpu.VMEM((4, 128), jnp.int32),
        pltpu.VMEM((RING, DIM, 128), jnp.float32),
        pltpu.VMEM((RING, DIM, 128), jnp.float32),
        pltpu.VMEM((DIM,), jnp.float32),
        pltpu.VMEM((LANES,), jnp.float32),
        pltpu.VMEM((ROWS_PER_W // 8, 128), jnp.float32),
        pltpu.VMEM((ROWS_PER_W,), jnp.float32),
        pltpu.SemaphoreType.DMA,
        pltpu.SemaphoreType.DMA,
        pltpu.SemaphoreType.DMA,
        pltpu.SemaphoreType.DMA,
    ]
    fn = pl.kernel(
        _sc_body,
        out_type=jax.ShapeDtypeStruct((BATCH,), jnp.float32),
        mesh=mesh,
        scratch_types=scratch,
        compiler_params=pltpu.CompilerParams(needs_layout_passes=False),
    )
    return fn(uidx2, iidx2, ttu, tti, w32, b16, dummy)


def kernel(user_indices, item_indices, user_table, item_table, W, b):
    uidx2 = user_indices.astype(jnp.int32).reshape(128, 128)
    iidx2 = item_indices.astype(jnp.int32).reshape(128, 128)
    ttu = user_table.T
    tti = item_table.T
    w32 = W.reshape(DIM)
    b16 = jnp.broadcast_to(b.reshape(1), (LANES,)).astype(jnp.float32)
    dummy = jnp.zeros((DIM, 128), jnp.float32)
    out = _run(uidx2, iidx2, ttu, tti, w32, b16, dummy)
    return out.reshape(BATCH, 1)


# speedup vs baseline: 4.3381x; 1.1801x over previous
import jax
import jax.numpy as jnp
from jax import lax
from jax.experimental import pallas as pl
from jax.experimental.pallas import tpu as pltpu
from jax.experimental.pallas import tpu_sc as plsc

BATCH = 16384
DIM = 32
LANES = 16
NUM_WORKERS = 32
ROWS_PER_W = BATCH // NUM_WORKERS
NGROUP = ROWS_PER_W // LANES
RING = 8


def _sc_body(uidx_hbm, iidx_hbm, ttu_hbm, tti_hbm, w_hbm, b_hbm, dummy_hbm,
             out_hbm,
             uidx_v, iidx_v, ubufw, ibufw, wv_v, bv_v, tbuf, outv,
             sem0, sem1, sem2, sem3, sem4, sem5, sem6, sem7):
    nc = 2
    wid = lax.axis_index("s") * nc + lax.axis_index("c")
    sems = (sem0, sem1, sem2, sem3, sem4, sem5, sem6, sem7)

    pltpu.sync_copy(uidx_hbm.at[pl.ds(wid * 4, 4)], uidx_v)
    pltpu.sync_copy(iidx_hbm.at[pl.ds(wid * 4, 4)], iidx_v)
    pltpu.sync_copy(w_hbm, wv_v)
    pltpu.sync_copy(b_hbm, bv_v)

    lanes = lax.iota(jnp.int32, LANES)
    wv0 = wv_v[pl.ds(0, LANES)]
    wv1 = wv_v[pl.ds(LANES, LANES)]

    def issue(slot, ur, ir):
        uc0 = pl.multiple_of((ur // 128) * 128, 128)
        ic0 = pl.multiple_of((ir // 128) * 128, 128)
        pltpu.async_copy(ttu_hbm.at[:, pl.ds(uc0, 128)],
                         ubufw.at[slot], sems[slot])
        pltpu.async_copy(tti_hbm.at[:, pl.ds(ic0, 128)],
                         ibufw.at[slot], sems[slot])

    def finish(slot, f, ur, ir):
        pltpu.make_async_copy(dummy_hbm, ubufw.at[slot], sems[slot]).wait()
        pltpu.make_async_copy(dummy_hbm, ibufw.at[slot], sems[slot]).wait()
        sv = jnp.full((LANES,), slot, jnp.int32)
        cu = jnp.full((LANES,), lax.rem(ur, 128), jnp.int32)
        ci = jnp.full((LANES,), lax.rem(ir, 128), jnp.int32)
        gu0 = plsc.load_gather(ubufw, [sv, lanes, cu])
        gu1 = plsc.load_gather(ubufw, [sv, lanes + LANES, cu])
        gi0 = plsc.load_gather(ibufw, [sv, lanes, ci])
        gi1 = plsc.load_gather(ibufw, [sv, lanes + LANES, ci])
        t = gu0 * gi0 * wv0 + gu1 * gi1 * wv1
        tbuf[f // 8, pl.ds(lax.rem(f, 8) * LANES, LANES)] = t

    def group_body(g, carry):
        puvec, pivec = carry
        f0 = g * LANES
        row = f0 // 128
        col = lax.rem(f0, 128)
        uvec = uidx_v[row, pl.ds(col, LANES)]
        ivec = iidx_v[row, pl.ds(col, LANES)]
        for k in range(LANES):
            slot = k % RING
            if k >= RING:
                finish(slot, f0 + k - RING, uvec[k - RING], ivec[k - RING])
            else:
                @pl.when(g > 0)
                def _fin(slot=slot, k=k, puvec=puvec, pivec=pivec):
                    finish(slot, f0 + k - RING,
                           puvec[LANES - RING + k], pivec[LANES - RING + k])
            issue(slot, uvec[k], ivec[k])
        return uvec, ivec

    zero16 = jnp.zeros((LANES,), jnp.int32)
    luvec, livec = lax.fori_loop(0, NGROUP, group_body, (zero16, zero16))

    for k in range(RING):
        finish((LANES - RING + k) % RING, ROWS_PER_W - RING + k,
               luvec[LANES - RING + k], livec[LANES - RING + k])

    def sig_body(j, carry):
        rows = j * LANES + lanes
        rr = rows // 8
        rc = lax.rem(rows, 8) * LANES
        x = bv_v[:]
        for l in range(LANES):
            x = x + plsc.load_gather(tbuf, [rr, rc + l])
        outv[pl.ds(j * LANES, LANES)] = 1.0 / (1.0 + jnp.exp(-x))
        return carry

    lax.fori_loop(0, ROWS_PER_W // LANES, sig_body, 0)

    pltpu.sync_copy(outv, out_hbm.at[pl.ds(wid * ROWS_PER_W, ROWS_PER_W)])


@jax.jit
def _run(uidx2, iidx2, ttu, tti, w32, b16, dummy):
    mesh = plsc.VectorSubcoreMesh(core_axis_name="c", subcore_axis_name="s",
                                  num_cores=2, num_subcores=16)
    scratch = [
        pltpu.VMEM((4, 128), jnp.int32),
        pltpu.VMEM((4, 128), jnp.int32),
        pltpu.VMEM((RING, DIM, 128), jnp.float32),
        pltpu.VMEM((RING, DIM, 128), jnp.float32),
        pltpu.VMEM((DIM,), jnp.float32),
        pltpu.VMEM((LANES,), jnp.float32),
        pltpu.VMEM((ROWS_PER_W // 8, 128), jnp.float32),
        pltpu.VMEM((ROWS_PER_W,), jnp.float32),
    ] + [pltpu.SemaphoreType.DMA] * RING
    fn = pl.kernel(
        _sc_body,
        out_type=jax.ShapeDtypeStruct((BATCH,), jnp.float32),
        mesh=mesh,
        scratch_types=scratch,
        compiler_params=pltpu.CompilerParams(needs_layout_passes=False),
    )
    return fn(uidx2, iidx2, ttu, tti, w32, b16, dummy)


def kernel(user_indices, item_indices, user_table, item_table, W, b):
    uidx2 = user_indices.astype(jnp.int32).reshape(128, 128)
    iidx2 = item_indices.astype(jnp.int32).reshape(128, 128)
    ttu = user_table.T
    tti = item_table.T
    w32 = W.reshape(DIM)
    b16 = jnp.broadcast_to(b.reshape(1), (LANES,)).astype(jnp.float32)
    dummy = jnp.zeros((DIM, 128), jnp.float32)
    out = _run(uidx2, iidx2, ttu, tti, w32, b16, dummy)
    return out.reshape(BATCH, 1)
